# baseline (speedup 1.0000x reference)
import functools

import jax
import jax.numpy as jnp
import numpy as np
from jax import lax
from jax.experimental import pallas as pl
from jax.experimental.pallas import tpu as pltpu
from jax.experimental.pallas import tpu_sc as plsc

_NUM_CLASSES = 80
_IOU_THRESHOLD = 0.5
_SCORE_THRESHOLD = 0.05
_MAX_BBOXES = 100
_B = 8
_N = 20000
_NEG = -1e30

_MAIN_BLK = 20480
_NPAD = 20480
_MAIN_BLKS = _NPAD // _MAIN_BLK
_CHUNK = 256
_NCHUNK = _NPAD // _CHUNK
_ACC_PAD = 112

_SC_CORES = 2


def _score_main_body(c_ref, p_ref, b_ref, o_ref, bo_ref):
    bo_ref[...] = b_ref[0].reshape(4, _MAIN_BLK // 128, 128)[None]
    c2 = c_ref[0]
    p = p_ref[0, 0, :]
    cmax = jnp.max(c2, axis=0)
    row = lax.broadcasted_iota(jnp.int32, c2.shape, 0)
    cls = jnp.min(jnp.where(c2 == cmax[None, :], row, _NUM_CLASSES + 47),
                  axis=0)
    sc = p * cls.astype(jnp.float32)
    y = jnp.where(sc > _SCORE_THRESHOLD, sc, _NEG)
    j = pl.program_id(1)
    limit = jnp.where(j == _MAIN_BLKS - 1,
                      _N - (_MAIN_BLKS - 1) * _MAIN_BLK, _MAIN_BLK)
    nid = lax.broadcasted_iota(jnp.int32, (_MAIN_BLK,), 0)
    y = jnp.where(nid < limit, y, _NEG)
    o_ref[...] = y.reshape(_MAIN_BLK // 128, 128)[None]


def _scores_all(ct, pt, bt):
    return pl.pallas_call(
        _score_main_body,
        grid=(_B, _MAIN_BLKS),
        in_specs=[
            pl.BlockSpec((1, _NUM_CLASSES, _MAIN_BLK),
                         lambda b, j: (b, 0, j)),
            pl.BlockSpec((1, 1, _MAIN_BLK), lambda b, j: (b, 0, j)),
            pl.BlockSpec((1, 4, _MAIN_BLK), lambda b, j: (b, 0, j)),
        ],
        out_specs=[
            pl.BlockSpec((1, _MAIN_BLK // 128, 128), lambda b, j: (b, j, 0)),
            pl.BlockSpec((1, 4, _MAIN_BLK // 128, 128),
                         lambda b, j: (b, 0, j, 0)),
        ],
        out_shape=[
            jax.ShapeDtypeStruct((_B, _NPAD // 128, 128), jnp.float32),
            jax.ShapeDtypeStruct((_B, 4, _NPAD // 128, 128), jnp.float32),
        ],
    )(ct, pt, bt)


def _nms_body(sm_hbm, box_hbm, pred_hbm, valid_hbm,
              s_v, box_v, cm_v, acc_v, stage_v, valid_v):
    wid = lax.axis_index("s") * _SC_CORES + lax.axis_index("c")

    @pl.when(wid < _B)
    def _run():
        b = wid
        pltpu.sync_copy(sm_hbm.at[b], s_v)
        pltpu.sync_copy(box_hbm.at[b], box_v)

        zf16 = jnp.zeros((16,), jnp.float32)
        for i in range(_MAX_BBOXES * 8 // 16):
            stage_v[pl.ds(i * 16, 16)] = zf16
        valid_v[...] = jnp.zeros((16,), jnp.int32)
        far = jnp.full((16,), 1e30, jnp.float32)
        for j in range(4):
            for g in range(_ACC_PAD // 16):
                acc_v[j, pl.ds(g * 16, 16)] = far

        iota = lax.broadcasted_iota(jnp.int32, (16,), 0)
        bigi = jnp.int32(1 << 30)

        def _set1(ref, pos, val):
            wbase = (pos // 16) * 16
            lane = pos % 16
            old = ref[pl.ds(wbase, 16)]
            ref[pl.ds(wbase, 16)] = jnp.where(iota == lane, val, old)

        def _chunk_max(ci):
            mv = s_v[2 * ci, pl.ds(0, 16)]
            for k in range(1, 16):
                mv = jnp.maximum(
                    mv, s_v[2 * ci + k // 8, pl.ds((k % 8) * 16, 16)])
            return jnp.max(mv)

        def _build_cm(ci, _):
            _set1(cm_v, ci, _chunk_max(ci))
            return 0
        lax.fori_loop(0, _NCHUNK, _build_cm, 0)

        def _cond(carry):
            cnt, go = carry
            return (go > 0) & (cnt < _MAX_BBOXES)

        def _body(carry):
            cnt, go = carry
            mv = cm_v[pl.ds(0, 16)]
            for j in range(1, _NCHUNK // 16):
                mv = jnp.maximum(mv, cm_v[pl.ds(j * 16, 16)])
            m = jnp.max(mv)
            cidx = jnp.full((16,), bigi, jnp.int32)
            for j in range(_NCHUNK // 16):
                v = cm_v[pl.ds(j * 16, 16)]
                cidx = jnp.minimum(cidx, jnp.where(v == m, iota + j * 16, bigi))
            cstar = jnp.min(cidx)

            eidx = jnp.full((16,), bigi, jnp.int32)
            for k in range(16):
                v = s_v[2 * cstar + k // 8, pl.ds((k % 8) * 16, 16)]
                eidx = jnp.minimum(eidx, jnp.where(v == m, iota + k * 16, bigi))
            off = jnp.min(eidx)
            idx = cstar * _CHUNK + off

            proceed = m > _SCORE_THRESHOLD

            z16 = jnp.zeros((16,), jnp.int32)
            bv = plsc.load_gather(
                box_v, [jnp.minimum(iota, 3), z16 + idx // 128,
                        z16 + idx % 128])
            v0 = bv[0]
            v1 = bv[1]
            v2 = bv[2]
            v3 = bv[3]
            by1 = jnp.minimum(v0, v2)
            bx1 = jnp.minimum(v1, v3)
            by2 = jnp.maximum(v0, v2)
            bx2 = jnp.maximum(v1, v3)
            area_b = (by2 - by1) * (bx2 - bx1)

            def _iou_group(g, acc_miou):
                ay1 = acc_v[0, pl.ds(g * 16, 16)]
                ax1 = acc_v[1, pl.ds(g * 16, 16)]
                ay2 = acc_v[2, pl.ds(g * 16, 16)]
                ax2 = acc_v[3, pl.ds(g * 16, 16)]
                iy1 = jnp.maximum(by1, ay1)
                ix1 = jnp.maximum(bx1, ax1)
                iy2 = jnp.minimum(by2, ay2)
                ix2 = jnp.minimum(bx2, ax2)
                inter = (jnp.maximum(iy2 - iy1, 0.0)
                         * jnp.maximum(ix2 - ix1, 0.0))
                area_a = (ay2 - ay1) * (ax2 - ax1)
                iou = inter / jnp.maximum(area_a + area_b - inter, 1e-8)
                return jnp.maximum(acc_miou, iou)
            miou = lax.fori_loop(0, (cnt + 15) // 16, _iou_group,
                                 jnp.zeros((16,), jnp.float32))
            supp = jnp.max(miou) > _IOU_THRESHOLD
            accept = proceed & jnp.logical_not(supp)

            @pl.when(accept)
            def _store():
                row = cnt * 8
                wbase = (row // 16) * 16
                off8 = row % 16
                old = stage_v[pl.ds(wbase, 16)]
                new = jnp.where(iota == off8 + 0,
                                jnp.minimum(jnp.maximum(v0, 0.0), 1.0), old)
                new = jnp.where(iota == off8 + 1,
                                jnp.minimum(jnp.maximum(v1, 0.0), 1.0), new)
                new = jnp.where(iota == off8 + 2,
                                jnp.minimum(jnp.maximum(v2, 0.0), 1.0), new)
                new = jnp.where(iota == off8 + 3,
                                jnp.minimum(jnp.maximum(v3, 0.0), 1.0), new)
                new = jnp.where(iota == off8 + 4, m, new)
                stage_v[pl.ds(wbase, 16)] = new
                g = (cnt // 16) * 16
                lane = cnt % 16
                for j, val in ((0, by1), (1, bx1), (2, by2), (3, bx2)):
                    olda = acc_v[j, pl.ds(g, 16)]
                    acc_v[j, pl.ds(g, 16)] = jnp.where(iota == lane, val, olda)

            @pl.when(proceed)
            def _suppress():
                koff = idx - cstar * _CHUNK
                srow = 2 * cstar + koff // 128
                scol = ((koff % 128) // 16) * 16
                lane = koff % 16
                oldv = s_v[srow, pl.ds(scol, 16)]
                s_v[srow, pl.ds(scol, 16)] = jnp.where(
                    iota == lane, _NEG, oldv)
                _set1(cm_v, cstar, _chunk_max(cstar))

            new_cnt = jnp.where(accept, cnt + 1, cnt)
            return new_cnt, jnp.where(proceed, 1, 0).astype(jnp.int32)

        cnt_final, _ = lax.while_loop(
            _cond, _body, (jnp.int32(0), jnp.int32(1)))
        valid_v[...] = jnp.where(iota == 0, cnt_final, 0)
        pltpu.sync_copy(stage_v, pred_hbm.at[b])
        pltpu.sync_copy(valid_v, valid_hbm.at[b])


_nms_kernel = functools.partial(
    pl.kernel,
    mesh=plsc.VectorSubcoreMesh(core_axis_name="c", subcore_axis_name="s"),
    compiler_params=pltpu.CompilerParams(needs_layout_passes=False),
    out_type=[
        jax.ShapeDtypeStruct((_B, _MAX_BBOXES * 8), jnp.float32),
        jax.ShapeDtypeStruct((_B, 16), jnp.int32),
    ],
    scratch_types=[
        pltpu.VMEM((_NPAD // 128, 128), jnp.float32),
        pltpu.VMEM((4, _NPAD // 128, 128), jnp.float32),
        pltpu.VMEM((_NCHUNK,), jnp.float32),
        pltpu.VMEM((4, _ACC_PAD), jnp.float32),
        pltpu.VMEM((_MAX_BBOXES * 8,), jnp.float32),
        pltpu.VMEM((16,), jnp.int32),
    ],
)(_nms_body)


def kernel(bbox20, p20, c20, training=False):
    ct = jnp.transpose(c20, (0, 2, 1))
    pt = jnp.transpose(p20, (0, 2, 1))
    bt = jnp.transpose(bbox20, (0, 2, 1))
    s_all, bbox = _scores_all(ct, pt, bt)
    pred_raw, valid_raw = _nms_kernel(s_all, bbox)
    pred = pred_raw.reshape(_B, _MAX_BBOXES, 8)[:, :, :6]
    valid = valid_raw[:, 0]
    return pred, valid


# speedup vs baseline: 1.0018x; 1.0018x over previous
import functools

import jax
import jax.numpy as jnp
import numpy as np
from jax import lax
from jax.experimental import pallas as pl
from jax.experimental.pallas import tpu as pltpu
from jax.experimental.pallas import tpu_sc as plsc

_NUM_CLASSES = 80
_IOU_THRESHOLD = 0.5
_SCORE_THRESHOLD = 0.05
_MAX_BBOXES = 100
_B = 8
_N = 20000
_NEG = -1e30

_MAIN_BLK = 20480
_NPAD = 20480
_MAIN_BLKS = _NPAD // _MAIN_BLK
_CHUNK = 256
_NCHUNK = _NPAD // _CHUNK
_ACC_PAD = 112

_SC_CORES = 2


def _score_main_body(c_ref, p_ref, b_ref, o_ref, bo_ref):
    bo_ref[...] = b_ref[0].reshape(4, _MAIN_BLK // 128, 128)[None]
    c2 = c_ref[0]
    p = p_ref[0, 0, :]
    cmax = jnp.max(c2, axis=0)
    row = lax.broadcasted_iota(jnp.int32, c2.shape, 0)
    cls = jnp.min(jnp.where(c2 == cmax[None, :], row, _NUM_CLASSES + 47),
                  axis=0)
    sc = p * cls.astype(jnp.float32)
    y = jnp.where(sc > _SCORE_THRESHOLD, sc, _NEG)
    j = pl.program_id(1)
    limit = jnp.where(j == _MAIN_BLKS - 1,
                      _N - (_MAIN_BLKS - 1) * _MAIN_BLK, _MAIN_BLK)
    nid = lax.broadcasted_iota(jnp.int32, (_MAIN_BLK,), 0)
    y = jnp.where(nid < limit, y, _NEG)
    o_ref[...] = y.reshape(_MAIN_BLK // 128, 128)[None]


def _scores_all(ct, pt, bt):
    return pl.pallas_call(
        _score_main_body,
        grid=(_B, _MAIN_BLKS),
        in_specs=[
            pl.BlockSpec((1, _NUM_CLASSES, _MAIN_BLK),
                         lambda b, j: (b, 0, j)),
            pl.BlockSpec((1, 1, _MAIN_BLK), lambda b, j: (b, 0, j)),
            pl.BlockSpec((1, 4, _MAIN_BLK), lambda b, j: (b, 0, j)),
        ],
        out_specs=[
            pl.BlockSpec((1, _MAIN_BLK // 128, 128), lambda b, j: (b, j, 0)),
            pl.BlockSpec((1, 4, _MAIN_BLK // 128, 128),
                         lambda b, j: (b, 0, j, 0)),
        ],
        out_shape=[
            jax.ShapeDtypeStruct((_B, _NPAD // 128, 128), jnp.float32),
            jax.ShapeDtypeStruct((_B, 4, _NPAD // 128, 128), jnp.float32),
        ],
    )(ct, pt, bt)


def _nms_body(sm_hbm, box_hbm, pred_hbm, valid_hbm,
              s_v, box_v, cm_v, acc_v, stage_v, valid_v):
    wid = lax.axis_index("s") * _SC_CORES + lax.axis_index("c")

    @pl.when(wid < _B)
    def _run():
        b = wid
        pltpu.sync_copy(sm_hbm.at[b], s_v)
        pltpu.sync_copy(box_hbm.at[b], box_v)

        zf16 = jnp.zeros((16,), jnp.float32)
        for i in range(_MAX_BBOXES * 8 // 16):
            stage_v[pl.ds(i * 16, 16)] = zf16
        valid_v[...] = jnp.zeros((16,), jnp.int32)
        far = jnp.full((16,), 1e30, jnp.float32)
        for j in range(4):
            for g in range(_ACC_PAD // 16):
                acc_v[j, pl.ds(g * 16, 16)] = far

        iota = lax.broadcasted_iota(jnp.int32, (16,), 0)
        bigi = jnp.int32(1 << 30)

        def _set1(ref, pos, val):
            wbase = (pos // 16) * 16
            lane = pos % 16
            old = ref[pl.ds(wbase, 16)]
            ref[pl.ds(wbase, 16)] = jnp.where(iota == lane, val, old)

        def _chunk_max(ci):
            mv = s_v[2 * ci, pl.ds(0, 16)]
            for k in range(1, 16):
                mv = jnp.maximum(
                    mv, s_v[2 * ci + k // 8, pl.ds((k % 8) * 16, 16)])
            return jnp.max(mv)

        def _build_cm(ci, _):
            _set1(cm_v, ci, _chunk_max(ci))
            return 0
        lax.fori_loop(0, _NCHUNK, _build_cm, 0)

        def _cond(carry):
            cnt, go = carry
            return (go > 0) & (cnt < _MAX_BBOXES)

        def _body(carry):
            cnt, go = carry
            mv = cm_v[pl.ds(0, 16)]
            for j in range(1, _NCHUNK // 16):
                mv = jnp.maximum(mv, cm_v[pl.ds(j * 16, 16)])
            m = jnp.max(mv)
            cidx = jnp.full((16,), bigi, jnp.int32)
            for j in range(_NCHUNK // 16):
                v = cm_v[pl.ds(j * 16, 16)]
                cidx = jnp.minimum(cidx, jnp.where(v == m, iota + j * 16, bigi))
            cstar = jnp.min(cidx)

            eidx = jnp.full((16,), bigi, jnp.int32)
            for k in range(16):
                v = s_v[2 * cstar + k // 8, pl.ds((k % 8) * 16, 16)]
                eidx = jnp.minimum(eidx, jnp.where(v == m, iota + k * 16, bigi))
            off = jnp.min(eidx)
            idx = cstar * _CHUNK + off

            proceed = m > _SCORE_THRESHOLD

            z16 = jnp.zeros((16,), jnp.int32)
            bv = plsc.load_gather(
                box_v, [jnp.minimum(iota, 3), z16 + idx // 128,
                        z16 + idx % 128])
            v0 = bv[0]
            v1 = bv[1]
            v2 = bv[2]
            v3 = bv[3]
            by1 = jnp.minimum(v0, v2)
            bx1 = jnp.minimum(v1, v3)
            by2 = jnp.maximum(v0, v2)
            bx2 = jnp.maximum(v1, v3)
            area_b = (by2 - by1) * (bx2 - bx1)

            miou = jnp.zeros((16,), jnp.float32)
            for g in range(_ACC_PAD // 16):
                ay1 = acc_v[0, pl.ds(g * 16, 16)]
                ax1 = acc_v[1, pl.ds(g * 16, 16)]
                ay2 = acc_v[2, pl.ds(g * 16, 16)]
                ax2 = acc_v[3, pl.ds(g * 16, 16)]
                iy1 = jnp.maximum(by1, ay1)
                ix1 = jnp.maximum(bx1, ax1)
                iy2 = jnp.minimum(by2, ay2)
                ix2 = jnp.minimum(bx2, ax2)
                inter = (jnp.maximum(iy2 - iy1, 0.0)
                         * jnp.maximum(ix2 - ix1, 0.0))
                area_a = (ay2 - ay1) * (ax2 - ax1)
                iou = inter / jnp.maximum(area_a + area_b - inter, 1e-8)
                miou = jnp.maximum(miou, iou)
            supp = jnp.max(miou) > _IOU_THRESHOLD
            accept = proceed & jnp.logical_not(supp)

            @pl.when(accept)
            def _store():
                row = cnt * 8
                wbase = (row // 16) * 16
                off8 = row % 16
                old = stage_v[pl.ds(wbase, 16)]
                new = jnp.where(iota == off8 + 0,
                                jnp.minimum(jnp.maximum(v0, 0.0), 1.0), old)
                new = jnp.where(iota == off8 + 1,
                                jnp.minimum(jnp.maximum(v1, 0.0), 1.0), new)
                new = jnp.where(iota == off8 + 2,
                                jnp.minimum(jnp.maximum(v2, 0.0), 1.0), new)
                new = jnp.where(iota == off8 + 3,
                                jnp.minimum(jnp.maximum(v3, 0.0), 1.0), new)
                new = jnp.where(iota == off8 + 4, m, new)
                stage_v[pl.ds(wbase, 16)] = new
                g = (cnt // 16) * 16
                lane = cnt % 16
                for j, val in ((0, by1), (1, bx1), (2, by2), (3, bx2)):
                    olda = acc_v[j, pl.ds(g, 16)]
                    acc_v[j, pl.ds(g, 16)] = jnp.where(iota == lane, val, olda)

            @pl.when(proceed)
            def _suppress():
                koff = idx - cstar * _CHUNK
                srow = 2 * cstar + koff // 128
                scol = ((koff % 128) // 16) * 16
                lane = koff % 16
                oldv = s_v[srow, pl.ds(scol, 16)]
                s_v[srow, pl.ds(scol, 16)] = jnp.where(
                    iota == lane, _NEG, oldv)
                _set1(cm_v, cstar, _chunk_max(cstar))

            new_cnt = jnp.where(accept, cnt + 1, cnt)
            return new_cnt, jnp.where(proceed, 1, 0).astype(jnp.int32)

        cnt_final, _ = lax.while_loop(
            _cond, _body, (jnp.int32(0), jnp.int32(1)))
        valid_v[...] = jnp.where(iota == 0, cnt_final, 0)
        pltpu.sync_copy(stage_v, pred_hbm.at[b])
        pltpu.sync_copy(valid_v, valid_hbm.at[b])


_nms_kernel = functools.partial(
    pl.kernel,
    mesh=plsc.VectorSubcoreMesh(core_axis_name="c", subcore_axis_name="s"),
    compiler_params=pltpu.CompilerParams(needs_layout_passes=False),
    out_type=[
        jax.ShapeDtypeStruct((_B, _MAX_BBOXES * 8), jnp.float32),
        jax.ShapeDtypeStruct((_B, 16), jnp.int32),
    ],
    scratch_types=[
        pltpu.VMEM((_NPAD // 128, 128), jnp.float32),
        pltpu.VMEM((4, _NPAD // 128, 128), jnp.float32),
        pltpu.VMEM((_NCHUNK,), jnp.float32),
        pltpu.VMEM((4, _ACC_PAD), jnp.float32),
        pltpu.VMEM((_MAX_BBOXES * 8,), jnp.float32),
        pltpu.VMEM((16,), jnp.int32),
    ],
)(_nms_body)


def kernel(bbox20, p20, c20, training=False):
    ct = jnp.transpose(c20, (0, 2, 1))
    pt = jnp.transpose(p20, (0, 2, 1))
    bt = jnp.transpose(bbox20, (0, 2, 1))
    s_all, bbox = _scores_all(ct, pt, bt)
    pred_raw, valid_raw = _nms_kernel(s_all, bbox)
    pred = pred_raw.reshape(_B, _MAX_BBOXES, 8)[:, :, :6]
    valid = valid_raw[:, 0]
    return pred, valid


# speedup vs baseline: 1.0033x; 1.0015x over previous
import functools

import jax
import jax.numpy as jnp
from jax import lax
from jax.experimental import pallas as pl
from jax.experimental.pallas import tpu as pltpu
from jax.experimental.pallas import tpu_sc as plsc

_NUM_CLASSES = 80
_IOU_THRESHOLD = 0.5
_SCORE_THRESHOLD = 0.05
_MAX_BBOXES = 100
_B = 8
_N = 20000
_NEG = -1e30

_MAIN_BLK = 20480
_NPAD = 20480
_MAIN_BLKS = _NPAD // _MAIN_BLK
_CHUNK = 256
_NCHUNK = _NPAD // _CHUNK
_ACC_PAD = 112

_SC_CORES = 2


def _score_main_body(c_ref, p_ref, b_ref, o_ref, bo_ref):
    bo_ref[...] = b_ref[0].reshape(4, _MAIN_BLK // 128, 128)[None]
    c2 = c_ref[0]
    p = p_ref[0, 0, :]
    cmax = jnp.max(c2, axis=0)
    row = lax.broadcasted_iota(jnp.int32, c2.shape, 0)
    cls = jnp.min(jnp.where(c2 == cmax[None, :], row, _NUM_CLASSES + 47),
                  axis=0)
    sc = p * cls.astype(jnp.float32)
    y = jnp.where(sc > _SCORE_THRESHOLD, sc, _NEG)
    j = pl.program_id(1)
    limit = jnp.where(j == _MAIN_BLKS - 1,
                      _N - (_MAIN_BLKS - 1) * _MAIN_BLK, _MAIN_BLK)
    nid = lax.broadcasted_iota(jnp.int32, (_MAIN_BLK,), 0)
    y = jnp.where(nid < limit, y, _NEG)
    o_ref[...] = y.reshape(_MAIN_BLK // 128, 128)[None]


def _scores_all(ct, pt, bt):
    return pl.pallas_call(
        _score_main_body,
        grid=(_B, _MAIN_BLKS),
        in_specs=[
            pl.BlockSpec((1, _NUM_CLASSES, _MAIN_BLK),
                         lambda b, j: (b, 0, j)),
            pl.BlockSpec((1, 1, _MAIN_BLK), lambda b, j: (b, 0, j)),
            pl.BlockSpec((1, 4, _MAIN_BLK), lambda b, j: (b, 0, j)),
        ],
        out_specs=[
            pl.BlockSpec((1, _MAIN_BLK // 128, 128), lambda b, j: (b, j, 0)),
            pl.BlockSpec((1, 4, _MAIN_BLK // 128, 128),
                         lambda b, j: (b, 0, j, 0)),
        ],
        out_shape=[
            jax.ShapeDtypeStruct((_B, _NPAD // 128, 128), jnp.float32),
            jax.ShapeDtypeStruct((_B, 4, _NPAD // 128, 128), jnp.float32),
        ],
    )(ct, pt, bt)


def _nms_body(sm_hbm, box_hbm, pred_hbm, valid_hbm,
              s_v, box_v, cm_v, acc_v, stage_v, valid_v):
    wid = lax.axis_index("s") * _SC_CORES + lax.axis_index("c")

    @pl.when(wid < _B)
    def _run():
        b = wid
        pltpu.sync_copy(sm_hbm.at[b], s_v)
        pltpu.sync_copy(box_hbm.at[b], box_v)

        zf16 = jnp.zeros((16,), jnp.float32)
        for i in range(_MAX_BBOXES * 8 // 16):
            stage_v[pl.ds(i * 16, 16)] = zf16
        valid_v[...] = jnp.zeros((16,), jnp.int32)
        far = jnp.full((16,), 1e30, jnp.float32)
        for j in range(4):
            for g in range(_ACC_PAD // 16):
                acc_v[j, pl.ds(g * 16, 16)] = far

        iota = lax.broadcasted_iota(jnp.int32, (16,), 0)
        bigi = jnp.int32(1 << 30)

        def _set1(ref, pos, val):
            wbase = (pos // 16) * 16
            lane = pos % 16
            old = ref[pl.ds(wbase, 16)]
            ref[pl.ds(wbase, 16)] = jnp.where(iota == lane, val, old)

        def _chunk_max(ci):
            mv = s_v[2 * ci, pl.ds(0, 16)]
            for k in range(1, 16):
                mv = jnp.maximum(
                    mv, s_v[2 * ci + k // 8, pl.ds((k % 8) * 16, 16)])
            return jnp.max(mv)

        def _build_cm(ci, _):
            _set1(cm_v, ci, _chunk_max(ci))
            return 0
        lax.fori_loop(0, _NCHUNK, _build_cm, 0)

        def _cond(carry):
            cnt, go = carry
            return (go > 0) & (cnt < _MAX_BBOXES)

        def _body(carry):
            cnt, go = carry
            mv = cm_v[pl.ds(0, 16)]
            for j in range(1, _NCHUNK // 16):
                mv = jnp.maximum(mv, cm_v[pl.ds(j * 16, 16)])
            m = jnp.max(mv)
            cidx = jnp.full((16,), bigi, jnp.int32)
            for j in range(_NCHUNK // 16):
                v = cm_v[pl.ds(j * 16, 16)]
                cidx = jnp.minimum(cidx, jnp.where(v == m, iota + j * 16, bigi))
            cstar = jnp.min(cidx)

            eidx = jnp.full((16,), bigi, jnp.int32)
            for k in range(16):
                v = s_v[2 * cstar + k // 8, pl.ds((k % 8) * 16, 16)]
                eidx = jnp.minimum(eidx, jnp.where(v == m, iota + k * 16, bigi))
            off = jnp.min(eidx)
            idx = cstar * _CHUNK + off

            proceed = m > _SCORE_THRESHOLD

            z16 = jnp.zeros((16,), jnp.int32)
            bv = plsc.load_gather(
                box_v, [jnp.minimum(iota, 3), z16 + idx // 128,
                        z16 + idx % 128])
            v0 = bv[0]
            v1 = bv[1]
            v2 = bv[2]
            v3 = bv[3]
            by1 = jnp.minimum(v0, v2)
            bx1 = jnp.minimum(v1, v3)
            by2 = jnp.maximum(v0, v2)
            bx2 = jnp.maximum(v1, v3)
            area_b = (by2 - by1) * (bx2 - bx1)

            miou = jnp.zeros((16,), jnp.float32)
            for g in range(_ACC_PAD // 16):
                ay1 = acc_v[0, pl.ds(g * 16, 16)]
                ax1 = acc_v[1, pl.ds(g * 16, 16)]
                ay2 = acc_v[2, pl.ds(g * 16, 16)]
                ax2 = acc_v[3, pl.ds(g * 16, 16)]
                iy1 = jnp.maximum(by1, ay1)
                ix1 = jnp.maximum(bx1, ax1)
                iy2 = jnp.minimum(by2, ay2)
                ix2 = jnp.minimum(bx2, ax2)
                inter = (jnp.maximum(iy2 - iy1, 0.0)
                         * jnp.maximum(ix2 - ix1, 0.0))
                area_a = (ay2 - ay1) * (ax2 - ax1)
                iou = inter / jnp.maximum(area_a + area_b - inter, 1e-8)
                miou = jnp.maximum(miou, iou)
            supp = jnp.max(miou) > _IOU_THRESHOLD
            accept = proceed & jnp.logical_not(supp)

            @pl.when(accept)
            def _store():
                row = cnt * 8
                wbase = (row // 16) * 16
                off8 = row % 16
                old = stage_v[pl.ds(wbase, 16)]
                new = jnp.where(iota == off8 + 0,
                                jnp.minimum(jnp.maximum(v0, 0.0), 1.0), old)
                new = jnp.where(iota == off8 + 1,
                                jnp.minimum(jnp.maximum(v1, 0.0), 1.0), new)
                new = jnp.where(iota == off8 + 2,
                                jnp.minimum(jnp.maximum(v2, 0.0), 1.0), new)
                new = jnp.where(iota == off8 + 3,
                                jnp.minimum(jnp.maximum(v3, 0.0), 1.0), new)
                new = jnp.where(iota == off8 + 4, m, new)
                stage_v[pl.ds(wbase, 16)] = new
                g = (cnt // 16) * 16
                lane = cnt % 16
                for j, val in ((0, by1), (1, bx1), (2, by2), (3, bx2)):
                    olda = acc_v[j, pl.ds(g, 16)]
                    acc_v[j, pl.ds(g, 16)] = jnp.where(iota == lane, val, olda)

            @pl.when(proceed)
            def _suppress():
                koff = idx - cstar * _CHUNK
                srow = 2 * cstar + koff // 128
                scol = ((koff % 128) // 16) * 16
                lane = koff % 16
                oldv = s_v[srow, pl.ds(scol, 16)]
                s_v[srow, pl.ds(scol, 16)] = jnp.where(
                    iota == lane, _NEG, oldv)
                _set1(cm_v, cstar, _chunk_max(cstar))

            new_cnt = jnp.where(accept, cnt + 1, cnt)
            return new_cnt, jnp.where(proceed, 1, 0).astype(jnp.int32)

        cnt_final, _ = lax.while_loop(
            _cond, _body, (jnp.int32(0), jnp.int32(1)))
        valid_v[...] = jnp.where(iota == 0, cnt_final, 0)
        pltpu.sync_copy(stage_v, pred_hbm.at[b])
        pltpu.sync_copy(valid_v, valid_hbm.at[b])


_nms_kernel = functools.partial(
    pl.kernel,
    mesh=plsc.VectorSubcoreMesh(core_axis_name="c", subcore_axis_name="s"),
    compiler_params=pltpu.CompilerParams(needs_layout_passes=False),
    out_type=[
        jax.ShapeDtypeStruct((_B, _MAX_BBOXES * 8), jnp.float32),
        jax.ShapeDtypeStruct((_B, 16), jnp.int32),
    ],
    scratch_types=[
        pltpu.VMEM((_NPAD // 128, 128), jnp.float32),
        pltpu.VMEM((4, _NPAD // 128, 128), jnp.float32),
        pltpu.VMEM((_NCHUNK,), jnp.float32),
        pltpu.VMEM((4, _ACC_PAD), jnp.float32),
        pltpu.VMEM((_MAX_BBOXES * 8,), jnp.float32),
        pltpu.VMEM((16,), jnp.int32),
    ],
)(_nms_body)


def kernel(bbox20, p20, c20, training=False):
    ct = jnp.transpose(c20, (0, 2, 1))
    pt = jnp.transpose(p20, (0, 2, 1))
    bt = jnp.transpose(bbox20, (0, 2, 1))
    s_all, bbox = _scores_all(ct, pt, bt)
    pred_raw, valid_raw = _nms_kernel(s_all, bbox)
    pred = pred_raw.reshape(_B, _MAX_BBOXES, 8)[:, :, :6]
    valid = valid_raw[:, 0]
    return pred, valid
